# baseline (speedup 1.0000x reference)
import functools

import jax
import jax.numpy as jnp
from jax import lax
from jax.experimental import pallas as pl
from jax.experimental.pallas import tpu as pltpu
from jax.experimental.pallas import tpu_sc as plsc

EMBED_DIM = 32
BATCH = 16384
N_ROWS = 1_000_000
ROWS_PER_LINE = 1
N_LINES = N_ROWS
LINE = 128
NUM_CORES = 2
NUM_SUBCORES = 16
NUM_WORKERS = NUM_CORES * NUM_SUBCORES
B_PER_W = BATCH // NUM_WORKERS
CHUNK = 128
N_CHUNKS = B_PER_W // CHUNK
LANES = 16
CGROUPS = CHUNK // LANES


def _sc_kernel(uid_hbm, mid_hbm, ut_hbm, mt_hbm, out_hbm,
               sidx_u, sidx_m, land_u, land_m, out_v, sem):
    wid = lax.axis_index("s") * NUM_CORES + lax.axis_index("c")
    base = wid * B_PER_W

    pltpu.sync_copy(uid_hbm.at[pl.ds(base, B_PER_W)], sidx_u)
    pltpu.sync_copy(mid_hbm.at[pl.ds(base, B_PER_W)], sidx_m)

    lane = lax.iota(jnp.int32, LANES)

    for c in range(N_CHUNKS):
        cu = pltpu.make_async_copy(
            ut_hbm.at[sidx_u.at[pl.ds(c * CHUNK, CHUNK)]], land_u, sem)
        cu.start()
        cm = pltpu.make_async_copy(
            mt_hbm.at[sidx_m.at[pl.ds(c * CHUNK, CHUNK)]], land_m, sem)
        cm.start()
        cu.wait()
        cm.wait()

        def body(g, _):
            slot = c * CHUNK + g * LANES
            rloc = g * LANES + lane
            acc = jnp.zeros((LANES,), jnp.float32)
            for d in range(EMBED_DIM):
                dvec = jnp.full((LANES,), d, jnp.int32)
                u = plsc.load_gather(land_u, [rloc, dvec])
                m = plsc.load_gather(land_m, [rloc, dvec])
                acc = acc + u * m
            out_v[pl.ds(slot, LANES)] = acc
            return 0

        lax.fori_loop(0, CGROUPS, body, 0)

    pltpu.sync_copy(out_v, out_hbm.at[pl.ds(base, B_PER_W)])


@jax.jit
def _run(user_id, movie_id, user_table, movie_table):
    k = functools.partial(
        pl.kernel,
        out_type=jax.ShapeDtypeStruct((BATCH,), jnp.float32),
        mesh=plsc.VectorSubcoreMesh(core_axis_name="c", subcore_axis_name="s"),
        compiler_params=pltpu.CompilerParams(
            needs_layout_passes=False, use_tc_tiling_on_sc=True),
        scratch_types=[
            pltpu.VMEM((B_PER_W,), jnp.int32),
            pltpu.VMEM((B_PER_W,), jnp.int32),
            pltpu.VMEM((CHUNK, LINE), jnp.float32),
            pltpu.VMEM((CHUNK, LINE), jnp.float32),
            pltpu.VMEM((B_PER_W,), jnp.float32),
            pltpu.SemaphoreType.DMA,
        ],
    )(_sc_kernel)
    return k(user_id.astype(jnp.int32), movie_id.astype(jnp.int32),
             jnp.pad(user_table, ((0, 0), (0, LINE - EMBED_DIM))),
             jnp.pad(movie_table, ((0, 0), (0, LINE - EMBED_DIM))))


def kernel(user_id, movie_id, user_table, movie_table):
    return _run(user_id, movie_id, user_table, movie_table)


# speedup vs baseline: 2.8797x; 2.8797x over previous
import functools

import jax
import jax.numpy as jnp
from jax import lax
from jax.experimental import pallas as pl
from jax.experimental.pallas import tpu as pltpu
from jax.experimental.pallas import tpu_sc as plsc

EMBED_DIM = 32
BATCH = 16384
N_ROWS = 1_000_000
ROWS_PER_LINE = 4
LINE = 128
NUM_CORES = 2
NUM_SUBCORES = 16
NUM_WORKERS = NUM_CORES * NUM_SUBCORES
B_PER_W = BATCH // NUM_WORKERS
CHUNK = 128
N_CHUNKS = B_PER_W // CHUNK
LANES = 16
CGROUPS = CHUNK // LANES

TBLK = 2048
TGRID = -(-N_ROWS // (4 * TBLK))
N_LINES = TGRID * TBLK
_LAST_IN_BLK = (N_ROWS - 1) // TBLK


def _tc_relayout_kernel(in0, in1, in2, in3, out_ref):
    x = jnp.concatenate(
        [in0[...], in1[...], in2[...], in3[...]], axis=0)
    out_ref[...] = jnp.transpose(x)


def _relayout(table_t):
    return pl.pallas_call(
        _tc_relayout_kernel,
        grid=(TGRID,),
        in_specs=[
            pl.BlockSpec(
                (EMBED_DIM, TBLK),
                functools.partial(
                    lambda n, r: (0, jnp.minimum(4 * n + r, _LAST_IN_BLK)),
                    r=r))
            for r in range(ROWS_PER_LINE)
        ],
        out_specs=pl.BlockSpec((TBLK, LINE), lambda n: (n, 0)),
        out_shape=jax.ShapeDtypeStruct((N_LINES, LINE), jnp.float32),
    )(table_t, table_t, table_t, table_t)


def _sc_kernel(uid_hbm, mid_hbm, ut_hbm, mt_hbm, out_hbm,
               sidx_u, sidx_m, qb_u, qb_m, land_u, land_m, out_v, sem):
    wid = lax.axis_index("s") * NUM_CORES + lax.axis_index("c")
    base = wid * B_PER_W

    pltpu.sync_copy(uid_hbm.at[pl.ds(base, B_PER_W)], sidx_u)
    pltpu.sync_copy(mid_hbm.at[pl.ds(base, B_PER_W)], sidx_m)

    def gen(g, _):
        vu = sidx_u[pl.ds(g * LANES, LANES)]
        vm = sidx_m[pl.ds(g * LANES, LANES)]
        qb_u[pl.ds(g * LANES, LANES)] = ((vu >> 13) << 11) + (vu & 2047)
        qb_m[pl.ds(g * LANES, LANES)] = ((vm >> 13) << 11) + (vm & 2047)
        return 0

    lax.fori_loop(0, B_PER_W // LANES, gen, 0)

    lane = lax.iota(jnp.int32, LANES)

    for c in range(N_CHUNKS):
        cu = pltpu.make_async_copy(
            ut_hbm.at[qb_u.at[pl.ds(c * CHUNK, CHUNK)]], land_u, sem)
        cu.start()
        cm = pltpu.make_async_copy(
            mt_hbm.at[qb_m.at[pl.ds(c * CHUNK, CHUNK)]], land_m, sem)
        cm.start()
        cu.wait()
        cm.wait()

        def body(g, _):
            slot = c * CHUNK + g * LANES
            su = ((sidx_u[pl.ds(slot, LANES)] >> 11) & 3) * EMBED_DIM
            sm = ((sidx_m[pl.ds(slot, LANES)] >> 11) & 3) * EMBED_DIM
            rloc = g * LANES + lane
            acc = jnp.zeros((LANES,), jnp.float32)
            for d in range(EMBED_DIM):
                u = plsc.load_gather(land_u, [rloc, su + d])
                m = plsc.load_gather(land_m, [rloc, sm + d])
                acc = acc + u * m
            out_v[pl.ds(slot, LANES)] = acc
            return 0

        lax.fori_loop(0, CGROUPS, body, 0)

    pltpu.sync_copy(out_v, out_hbm.at[pl.ds(base, B_PER_W)])


@jax.jit
def _run(user_id, movie_id, user_table, movie_table):
    ut2 = _relayout(user_table.T)
    mt2 = _relayout(movie_table.T)
    k = functools.partial(
        pl.kernel,
        out_type=jax.ShapeDtypeStruct((BATCH,), jnp.float32),
        mesh=plsc.VectorSubcoreMesh(core_axis_name="c", subcore_axis_name="s"),
        compiler_params=pltpu.CompilerParams(
            needs_layout_passes=False, use_tc_tiling_on_sc=True),
        scratch_types=[
            pltpu.VMEM((B_PER_W,), jnp.int32),
            pltpu.VMEM((B_PER_W,), jnp.int32),
            pltpu.VMEM((B_PER_W,), jnp.int32),
            pltpu.VMEM((B_PER_W,), jnp.int32),
            pltpu.VMEM((CHUNK, LINE), jnp.float32),
            pltpu.VMEM((CHUNK, LINE), jnp.float32),
            pltpu.VMEM((B_PER_W,), jnp.float32),
            pltpu.SemaphoreType.DMA,
        ],
    )(_sc_kernel)
    return k(user_id.astype(jnp.int32), movie_id.astype(jnp.int32), ut2, mt2)


def kernel(user_id, movie_id, user_table, movie_table):
    return _run(user_id, movie_id, user_table, movie_table)


# speedup vs baseline: 3.8096x; 1.3229x over previous
import functools

import jax
import jax.numpy as jnp
from jax import lax
from jax.experimental import pallas as pl
from jax.experimental.pallas import tpu as pltpu
from jax.experimental.pallas import tpu_sc as plsc

EMBED_DIM = 32
BATCH = 16384
N_ROWS = 1_000_000
ROWS_PER_LINE = 4
LINE = 128
NUM_CORES = 2
NUM_SUBCORES = 16
NUM_WORKERS = NUM_CORES * NUM_SUBCORES
B_PER_W = BATCH // NUM_WORKERS
CHUNK = 128
N_CHUNKS = B_PER_W // CHUNK
LANES = 16
CGROUPS = CHUNK // LANES

TBLK = 4096
TGRID = -(-N_ROWS // (4 * TBLK))
N_LINES = TGRID * TBLK
_LAST_IN_BLK = (N_ROWS - 1) // TBLK


def _tc_relayout_kernel(in0, in1, in2, in3, out_ref):
    x = jnp.concatenate(
        [in0[...], in1[...], in2[...], in3[...]], axis=0)
    out_ref[...] = jnp.transpose(x)


def _relayout(table_t):
    return pl.pallas_call(
        _tc_relayout_kernel,
        grid=(TGRID,),
        in_specs=[
            pl.BlockSpec(
                (EMBED_DIM, TBLK),
                functools.partial(
                    lambda n, r: (0, jnp.minimum(4 * n + r, _LAST_IN_BLK)),
                    r=r))
            for r in range(ROWS_PER_LINE)
        ],
        out_specs=pl.BlockSpec((TBLK, LINE), lambda n: (n, 0)),
        out_shape=jax.ShapeDtypeStruct((N_LINES, LINE), jnp.float32),
    )(table_t, table_t, table_t, table_t)


def _sc_kernel(uid_hbm, mid_hbm, ut_hbm, mt_hbm, out_hbm,
               sidx_u, sidx_m, qb_u, qb_m, land_u, land_m, out_v, sem):
    wid = lax.axis_index("s") * NUM_CORES + lax.axis_index("c")
    base = wid * B_PER_W

    pltpu.sync_copy(uid_hbm.at[pl.ds(base, B_PER_W)], sidx_u)
    pltpu.sync_copy(mid_hbm.at[pl.ds(base, B_PER_W)], sidx_m)

    def gen(g, _):
        vu = sidx_u[pl.ds(g * LANES, LANES)]
        vm = sidx_m[pl.ds(g * LANES, LANES)]
        qb_u[pl.ds(g * LANES, LANES)] = ((vu >> 14) << 12) + (vu & 4095)
        qb_m[pl.ds(g * LANES, LANES)] = ((vm >> 14) << 12) + (vm & 4095)
        return 0

    lax.fori_loop(0, B_PER_W // LANES, gen, 0)

    lane = lax.iota(jnp.int32, LANES)

    for c in range(N_CHUNKS):
        cu = pltpu.make_async_copy(
            ut_hbm.at[qb_u.at[pl.ds(c * CHUNK, CHUNK)]], land_u, sem)
        cu.start()
        cm = pltpu.make_async_copy(
            mt_hbm.at[qb_m.at[pl.ds(c * CHUNK, CHUNK)]], land_m, sem)
        cm.start()
        cu.wait()
        cm.wait()

        def body(g, _):
            slot = c * CHUNK + g * LANES
            su = ((sidx_u[pl.ds(slot, LANES)] >> 12) & 3) * EMBED_DIM
            sm = ((sidx_m[pl.ds(slot, LANES)] >> 12) & 3) * EMBED_DIM
            rloc = g * LANES + lane
            acc = jnp.zeros((LANES,), jnp.float32)
            for d in range(EMBED_DIM):
                u = plsc.load_gather(land_u, [rloc, su + d])
                m = plsc.load_gather(land_m, [rloc, sm + d])
                acc = acc + u * m
            out_v[pl.ds(slot, LANES)] = acc
            return 0

        lax.fori_loop(0, CGROUPS, body, 0)

    pltpu.sync_copy(out_v, out_hbm.at[pl.ds(base, B_PER_W)])


@jax.jit
def _run(user_id, movie_id, user_table, movie_table):
    ut2 = _relayout(user_table.T)
    mt2 = _relayout(movie_table.T)
    k = functools.partial(
        pl.kernel,
        out_type=jax.ShapeDtypeStruct((BATCH,), jnp.float32),
        mesh=plsc.VectorSubcoreMesh(core_axis_name="c", subcore_axis_name="s"),
        compiler_params=pltpu.CompilerParams(
            needs_layout_passes=False, use_tc_tiling_on_sc=True),
        scratch_types=[
            pltpu.VMEM((B_PER_W,), jnp.int32),
            pltpu.VMEM((B_PER_W,), jnp.int32),
            pltpu.VMEM((B_PER_W,), jnp.int32),
            pltpu.VMEM((B_PER_W,), jnp.int32),
            pltpu.VMEM((CHUNK, LINE), jnp.float32),
            pltpu.VMEM((CHUNK, LINE), jnp.float32),
            pltpu.VMEM((B_PER_W,), jnp.float32),
            pltpu.SemaphoreType.DMA,
        ],
    )(_sc_kernel)
    return k(user_id.astype(jnp.int32), movie_id.astype(jnp.int32), ut2, mt2)


def kernel(user_id, movie_id, user_table, movie_table):
    return _run(user_id, movie_id, user_table, movie_table)


# speedup vs baseline: 4.3527x; 1.1426x over previous
import functools

import jax
import jax.numpy as jnp
from jax import lax
from jax.experimental import pallas as pl
from jax.experimental.pallas import tpu as pltpu
from jax.experimental.pallas import tpu_sc as plsc

EMBED_DIM = 32
BATCH = 16384
N_ROWS = 1_000_000
ROWS_PER_LINE = 4
LINE = 128
NUM_CORES = 2
NUM_SUBCORES = 16
NUM_WORKERS = NUM_CORES * NUM_SUBCORES
B_PER_W = BATCH // NUM_WORKERS
CHUNK = 128
N_CHUNKS = B_PER_W // CHUNK
LANES = 16
CGROUPS = CHUNK // LANES

TBLK = 8192
TGRID = -(-N_ROWS // (4 * TBLK))
N_LINES = TGRID * TBLK
_LAST_IN_BLK = (N_ROWS - 1) // TBLK


def _tc_relayout_kernel(in0, in1, in2, in3, out_ref):
    x = jnp.concatenate(
        [in0[...], in1[...], in2[...], in3[...]], axis=0)
    out_ref[...] = jnp.transpose(x)


def _relayout(table_t):
    return pl.pallas_call(
        _tc_relayout_kernel,
        grid=(TGRID,),
        in_specs=[
            pl.BlockSpec(
                (EMBED_DIM, TBLK),
                functools.partial(
                    lambda n, r: (0, jnp.minimum(4 * n + r, _LAST_IN_BLK)),
                    r=r))
            for r in range(ROWS_PER_LINE)
        ],
        out_specs=pl.BlockSpec((TBLK, LINE), lambda n: (n, 0)),
        out_shape=jax.ShapeDtypeStruct((N_LINES, LINE), jnp.float32),
    )(table_t, table_t, table_t, table_t)


def _sc_kernel(uid_hbm, mid_hbm, ut_hbm, mt_hbm, out_hbm,
               sidx_u, sidx_m, qb_u, qb_m, land_u, land_m, out_v, sem):
    wid = lax.axis_index("s") * NUM_CORES + lax.axis_index("c")
    base = wid * B_PER_W

    pltpu.sync_copy(uid_hbm.at[pl.ds(base, B_PER_W)], sidx_u)
    pltpu.sync_copy(mid_hbm.at[pl.ds(base, B_PER_W)], sidx_m)

    def gen(g, _):
        vu = sidx_u[pl.ds(g * LANES, LANES)]
        vm = sidx_m[pl.ds(g * LANES, LANES)]
        qb_u[pl.ds(g * LANES, LANES)] = ((vu >> 15) << 13) + (vu & 8191)
        qb_m[pl.ds(g * LANES, LANES)] = ((vm >> 15) << 13) + (vm & 8191)
        return 0

    lax.fori_loop(0, B_PER_W // LANES, gen, 0)

    lane = lax.iota(jnp.int32, LANES)

    for c in range(N_CHUNKS):
        cu = pltpu.make_async_copy(
            ut_hbm.at[qb_u.at[pl.ds(c * CHUNK, CHUNK)]], land_u, sem)
        cu.start()
        cm = pltpu.make_async_copy(
            mt_hbm.at[qb_m.at[pl.ds(c * CHUNK, CHUNK)]], land_m, sem)
        cm.start()
        cu.wait()
        cm.wait()

        def body(g, _):
            slot = c * CHUNK + g * LANES
            su = ((sidx_u[pl.ds(slot, LANES)] >> 13) & 3) * EMBED_DIM
            sm = ((sidx_m[pl.ds(slot, LANES)] >> 13) & 3) * EMBED_DIM
            rloc = g * LANES + lane
            acc = jnp.zeros((LANES,), jnp.float32)
            for d in range(EMBED_DIM):
                u = plsc.load_gather(land_u, [rloc, su + d])
                m = plsc.load_gather(land_m, [rloc, sm + d])
                acc = acc + u * m
            out_v[pl.ds(slot, LANES)] = acc
            return 0

        lax.fori_loop(0, CGROUPS, body, 0)

    pltpu.sync_copy(out_v, out_hbm.at[pl.ds(base, B_PER_W)])


@jax.jit
def _run(user_id, movie_id, user_table, movie_table):
    ut2 = _relayout(user_table.T)
    mt2 = _relayout(movie_table.T)
    k = functools.partial(
        pl.kernel,
        out_type=jax.ShapeDtypeStruct((BATCH,), jnp.float32),
        mesh=plsc.VectorSubcoreMesh(core_axis_name="c", subcore_axis_name="s"),
        compiler_params=pltpu.CompilerParams(
            needs_layout_passes=False, use_tc_tiling_on_sc=True),
        scratch_types=[
            pltpu.VMEM((B_PER_W,), jnp.int32),
            pltpu.VMEM((B_PER_W,), jnp.int32),
            pltpu.VMEM((B_PER_W,), jnp.int32),
            pltpu.VMEM((B_PER_W,), jnp.int32),
            pltpu.VMEM((CHUNK, LINE), jnp.float32),
            pltpu.VMEM((CHUNK, LINE), jnp.float32),
            pltpu.VMEM((B_PER_W,), jnp.float32),
            pltpu.SemaphoreType.DMA,
        ],
    )(_sc_kernel)
    return k(user_id.astype(jnp.int32), movie_id.astype(jnp.int32), ut2, mt2)


def kernel(user_id, movie_id, user_table, movie_table):
    return _run(user_id, movie_id, user_table, movie_table)
